# baseline (speedup 1.0000x reference)
import jax
import jax.numpy as jnp
from jax.experimental import pallas as pl
from jax.experimental.pallas import tpu as pltpu

BATCH = 1024
FEATS = 4096
NXQ = 4
BM = BATCH // NXQ
BN = 256


def _matmul_body(x0, x1, x2, x3, wa_ref, wb_ref, b_ref, o_ref):
    dn = (((1,), (1,)), ((), ()))

    def dot(x_ref, w_ref):
        return jax.lax.dot_general(
            x_ref[...], w_ref[...], dimension_numbers=dn,
            preferred_element_type=jnp.float32,
            precision=jax.lax.Precision.DEFAULT,
        )

    for q, x_ref in enumerate((x0, x1, x2, x3)):
        rows = pl.ds(q * BM, BM)
        o_ref[rows, :BN] = dot(x_ref, wa_ref) + b_ref[:, :BN]
        o_ref[rows, BN:] = dot(x_ref, wb_ref) + b_ref[:, BN:]


def kernel(x, weight, bias):
    bias2d = bias.reshape(1, FEATS)
    grid = (FEATS // (2 * BN),)
    xq_spec = [
        pl.BlockSpec((BM, FEATS), (lambda q: (lambda j: (q, 0)))(q))
        for q in range(NXQ)
    ]
    return pl.pallas_call(
        _matmul_body,
        grid=grid,
        in_specs=xq_spec + [
            pl.BlockSpec((BN, FEATS), lambda j: (2 * j, 0)),
            pl.BlockSpec((BN, FEATS), lambda j: (2 * j + 1, 0)),
            pl.BlockSpec((1, 2 * BN), lambda j: (0, j)),
        ],
        out_specs=pl.BlockSpec((BATCH, 2 * BN), lambda j: (0, j)),
        out_shape=jax.ShapeDtypeStruct((BATCH, FEATS), jnp.float32),
        compiler_params=pltpu.CompilerParams(
            dimension_semantics=("arbitrary",),
        ),
    )(x, x, x, x, weight, weight, bias2d)


# speedup vs baseline: 1.0292x; 1.0292x over previous
import jax
import jax.numpy as jnp
from jax.experimental import pallas as pl
from jax.experimental.pallas import tpu as pltpu

BATCH = 1024
FEATS = 4096
BN = 256


def _matmul_body(x_ref, wa_ref, wb_ref, b_ref, o_ref):
    x = x_ref[...]
    dn = (((1,), (1,)), ((), ()))

    def dot(w_ref):
        return jax.lax.dot_general(
            x, w_ref[...], dimension_numbers=dn,
            preferred_element_type=jnp.float32,
            precision=jax.lax.Precision.DEFAULT,
        )

    o_ref[:, :BN] = dot(wa_ref) + b_ref[:, :BN]
    o_ref[:, BN:] = dot(wb_ref) + b_ref[:, BN:]


def kernel(x, weight, bias):
    bias2d = bias.reshape(1, FEATS)
    grid = (FEATS // (2 * BN),)
    return pl.pallas_call(
        _matmul_body,
        grid=grid,
        in_specs=[
            pl.BlockSpec((BATCH, FEATS), lambda j: (0, 0)),
            pl.BlockSpec((BN, FEATS), lambda j: (2 * j, 0)),
            pl.BlockSpec((BN, FEATS), lambda j: (2 * j + 1, 0)),
            pl.BlockSpec((1, 2 * BN), lambda j: (0, j)),
        ],
        out_specs=pl.BlockSpec((BATCH, 2 * BN), lambda j: (0, j)),
        out_shape=jax.ShapeDtypeStruct((BATCH, FEATS), jnp.float32),
        compiler_params=pltpu.CompilerParams(
            dimension_semantics=("arbitrary",),
        ),
    )(x, weight, weight, bias2d)
